# baseline (speedup 1.0000x reference)
import functools

import jax
import jax.numpy as jnp
from jax import lax
from jax.experimental import pallas as pl
from jax.experimental.pallas import tpu as pltpu
from jax.experimental.pallas import tpu_sc as plsc

B = 4096
D = 32
NC = 2
NS = 16
NW = NC * NS
CHUNK = B // NW
L = 16


_GDN = lax.GatherDimensionNumbers(
    offset_dims=(), collapsed_slice_dims=(0,), start_index_map=(0,))


def _permute(v, idx):
    return lax.gather(v, idx[:, None], _GDN, slice_sizes=(1,),
                      mode=lax.GatherScatterMode.PROMISE_IN_BOUNDS)


def _lane_sum(v):
    for sh in (1, 2, 4, 8):
        idx = lax.iota(jnp.int32, L) ^ sh
        v = v + _permute(v, idx)
    return v


def _sc_body(ls_hbm, rs_hbm, w_hbm, b_hbm, s_hbm, t_hbm,
             idx_l, idx_r, lw, rw, lb, rb, s_chunk, t_chunk,
             sem0, sem1, sem2, sem3):
    wid = lax.axis_index("s") * NC + lax.axis_index("c")
    base = wid * CHUNK
    pltpu.sync_copy(ls_hbm.at[pl.ds(base, CHUNK)], idx_l)
    pltpu.sync_copy(rs_hbm.at[pl.ds(base, CHUNK)], idx_r)
    cl = pltpu.async_copy(w_hbm.at[idx_l], lw, sem0)
    cr = pltpu.async_copy(w_hbm.at[idx_r], rw, sem1)
    cbl = pltpu.async_copy(b_hbm.at[idx_l], lb, sem2)
    cbr = pltpu.async_copy(b_hbm.at[idx_r], rb, sem3)
    cl.wait()
    cr.wait()
    cbl.wait()
    cbr.wait()
    iota = lax.iota(jnp.int32, L)
    for k in range(CHUNK // L):
        acc = jnp.zeros((L,), jnp.float32)
        for j in range(L):
            r = k * L + j
            p = (lw[r, pl.ds(0, L)] * rw[r, pl.ds(0, L)] +
                 lw[r, pl.ds(L, L)] * rw[r, pl.ds(L, L)])
            acc = jnp.where(iota == j, _lane_sum(p), acc)
        s_chunk[pl.ds(k * L, L)] = acc
        tb = lb[pl.ds(k * L, L)] + rb[pl.ds(k * L, L)]
        t_chunk[pl.ds(k * L, L)] = tb
    pltpu.sync_copy(s_chunk, s_hbm.at[pl.ds(base, CHUNK)])
    pltpu.sync_copy(t_chunk, t_hbm.at[pl.ds(base, CHUNK)])


_sc_dot = functools.partial(
    pl.kernel,
    out_type=(jax.ShapeDtypeStruct((B,), jnp.float32),
              jax.ShapeDtypeStruct((B,), jnp.float32)),
    mesh=plsc.VectorSubcoreMesh(core_axis_name="c", subcore_axis_name="s"),
    scratch_types=[
        pltpu.VMEM((CHUNK,), jnp.int32),
        pltpu.VMEM((CHUNK,), jnp.int32),
        pltpu.VMEM((CHUNK, D), jnp.float32),
        pltpu.VMEM((CHUNK, D), jnp.float32),
        pltpu.VMEM((CHUNK,), jnp.float32),
        pltpu.VMEM((CHUNK,), jnp.float32),
        pltpu.VMEM((CHUNK,), jnp.float32),
        pltpu.VMEM((CHUNK,), jnp.float32),
        pltpu.SemaphoreType.DMA,
        pltpu.SemaphoreType.DMA,
        pltpu.SemaphoreType.DMA,
        pltpu.SemaphoreType.DMA,
    ],
    compiler_params=pltpu.CompilerParams(use_tc_tiling_on_sc=False),
)(_sc_body)


ROW_BLK = 256


def _bcast_body(t_ref, s_ref, out_ref):
    t = t_ref[0, 0, :]
    s = s_ref[0, :]
    out_ref[...] = t[:, None] + s[None, :]


_bcast = pl.pallas_call(
    _bcast_body,
    out_shape=jax.ShapeDtypeStruct((B, B), jnp.float32),
    grid=(B // ROW_BLK,),
    in_specs=[
        pl.BlockSpec((1, 1, ROW_BLK), lambda i: (i, 0, 0)),
        pl.BlockSpec((1, B), lambda i: (0, 0)),
    ],
    out_specs=pl.BlockSpec((ROW_BLK, B), lambda i: (i, 0)),
)


def kernel(ls, rs, W, b):
    s, t = _sc_dot(ls, rs, W, b.reshape(-1))
    return _bcast(t.reshape(B // ROW_BLK, 1, ROW_BLK), s.reshape(1, B))
